# baseline (speedup 1.0000x reference)
import functools

import numpy as np
import jax
import jax.numpy as jnp
from jax import lax
from jax.experimental import pallas as pl
from jax.experimental.pallas import tpu as pltpu
from jax.experimental.pallas import tpu_sc as plsc

N = 100000
MAXDEG = 32
F = 128
H = 256
S1 = 10
S2 = 25
B = 1024
C = 41

_PERM0 = np.random.RandomState(0).permutation(MAXDEG)
_PERM1 = np.random.RandomState(1).permutation(MAXDEG)

_SEL = np.zeros((MAXDEG, 128), np.float32)
for _i in range(S2):
    _SEL[_PERM0[_i], _i] = 1.0
for _j in range(S1):
    _SEL[_PERM1[_j], 32 + _j] = 1.0

_NC = 2
_NS = 16
_NW = _NC * _NS


def _sc_mesh():
    return plsc.VectorSubcoreMesh(core_axis_name="c", subcore_axis_name="s")


@functools.partial(jax.jit, static_argnums=(2, 3, 4))
def _sc_gather(table, idx, M, D, R):
    m = M // _NW
    nch = m // R
    dtype = table.dtype

    @functools.partial(
        pl.kernel,
        mesh=_sc_mesh(),
        out_type=jax.ShapeDtypeStruct((M, D), dtype),
        scratch_types=[
            pltpu.VMEM((R,), jnp.int32),
            pltpu.VMEM((R,), jnp.int32),
            pltpu.VMEM((R, D), dtype),
            pltpu.VMEM((R, D), dtype),
            pltpu.SemaphoreType.DMA,
            pltpu.SemaphoreType.DMA,
        ],
    )
    def k(table_hbm, idx_hbm, out_hbm, ixa, ixb, bufa, bufb, gsem, ssem):
        wid = lax.axis_index("s") * _NC + lax.axis_index("c")
        base = wid * m

        def pf(c, ix, buf):
            pltpu.sync_copy(idx_hbm.at[pl.ds(base + c * R, R)], ix)
            pltpu.make_async_copy(table_hbm.at[ix], buf, gsem).start()

        def gwait(ix, buf):
            pltpu.make_async_copy(table_hbm.at[ix], buf, gsem).wait()

        def sto(c, buf):
            return pltpu.make_async_copy(
                buf, out_hbm.at[pl.ds(base + c * R, R)], ssem)

        if nch == 1:
            pf(0, ixa, bufa)
            gwait(ixa, bufa)
            pltpu.sync_copy(bufa, out_hbm.at[pl.ds(base, R)])
            return

        nit = nch // 2
        pf(0, ixa, bufa)

        def body(i, carry):
            a = 2 * i
            b = a + 1

            @pl.when(i > 0)
            def _():
                sto(b - 2, bufb).wait()

            pf(b, ixb, bufb)
            gwait(ixa, bufa)
            sto(a, bufa).start()
            gwait(ixb, bufb)
            sto(a, bufa).wait()

            @pl.when(i < nit - 1)
            def _():
                pf(a + 2, ixa, bufa)

            sto(b, bufb).start()
            return carry

        lax.fori_loop(0, nit, body, 0)
        sto(nch - 1, bufb).wait()

    return k(table, idx)


_CT = 4
_RC = _CT * S1


@functools.partial(jax.jit, static_argnums=(2,))
def _sc_gather_sum(table, idx, M2):
    m = M2 // _NW
    nch = m // _CT
    nit = nch // 2

    @functools.partial(
        pl.kernel,
        mesh=_sc_mesh(),
        out_type=jax.ShapeDtypeStruct((M2, F), jnp.float32),
        scratch_types=[
            pltpu.VMEM((_RC,), jnp.int32),
            pltpu.VMEM((_RC,), jnp.int32),
            pltpu.VMEM((_RC, F), jnp.float32),
            pltpu.VMEM((_RC, F), jnp.float32),
            pltpu.VMEM((_CT, F), jnp.float32),
            pltpu.VMEM((_CT, F), jnp.float32),
            pltpu.SemaphoreType.DMA,
            pltpu.SemaphoreType.DMA,
        ],
    )
    def k(table_hbm, idx_hbm, out_hbm, ixa, ixb, bufa, bufb, acca, accb,
          gsem, ssem):
        wid = lax.axis_index("s") * _NC + lax.axis_index("c")
        tbase = wid * m

        def pf(c, ix, buf):
            pltpu.sync_copy(
                idx_hbm.at[pl.ds((tbase + c * _CT) * S1, _RC)], ix)
            pltpu.make_async_copy(table_hbm.at[ix], buf, gsem).start()

        def gwait(ix, buf):
            pltpu.make_async_copy(table_hbm.at[ix], buf, gsem).wait()

        def sto(c, acc):
            return pltpu.make_async_copy(
                acc, out_hbm.at[pl.ds(tbase + c * _CT, _CT)], ssem)

        def reduce(buf, acc):
            for t in range(_CT):
                for v in range(F // 16):
                    sl = pl.ds(v * 16, 16)
                    a = buf[t * S1, sl]
                    for s in range(1, S1):
                        a = a + buf[t * S1 + s, sl]
                    acc[t, sl] = a

        pf(0, ixa, bufa)

        def body(i, carry):
            a = 2 * i
            b = a + 1
            pf(b, ixb, bufb)
            gwait(ixa, bufa)

            @pl.when(i > 0)
            def _():
                sto(a - 2, acca).wait()

            reduce(bufa, acca)
            sto(a, acca).start()

            @pl.when(i < nit - 1)
            def _():
                pf(a + 2, ixa, bufa)

            gwait(ixb, bufb)

            @pl.when(i > 0)
            def _():
                sto(b - 2, accb).wait()

            reduce(bufb, accb)
            sto(b, accb).start()
            return carry

        lax.fori_loop(0, nit, body, 0)
        sto(nch - 2, acca).wait()
        sto(nch - 1, accb).wait()

    return k(table, idx)


_BRB = 2000


def _build_body(adj_ref, sel_ref, out_ref):
    af = adj_ref[...].astype(jnp.float32)
    p = jnp.dot(af, sel_ref[...], preferred_element_type=jnp.float32,
                precision=lax.Precision.HIGHEST)
    out_ref[...] = jnp.round(p).astype(jnp.int32)


@jax.jit
def _tc_build_adjp(adj, sel):
    return pl.pallas_call(
        _build_body,
        grid=(N // _BRB,),
        in_specs=[
            pl.BlockSpec((_BRB, MAXDEG), lambda i: (i, 0)),
            pl.BlockSpec((MAXDEG, 128), lambda i: (0, 0)),
        ],
        out_specs=pl.BlockSpec((_BRB, 128), lambda i: (i, 0)),
        out_shape=jax.ShapeDtypeStruct((N, 128), jnp.int32),
    )(adj, sel)


_BLK = 800
_G = _BLK // S2


def _tc1_body(gh0_ref, gs_ref, wsB_ref, bsB_ref, wnB_ref, bnB_ref,
              x1_ref, n0_ref):
    gh0 = gh0_ref[...]
    gs = gs_ref[...] * (1.0 / S1)
    hs = jnp.maximum(
        jnp.dot(gh0, wsB_ref[...], preferred_element_type=jnp.float32)
        + bsB_ref[...], 0.0)
    hn = jnp.maximum(
        jnp.dot(gs, wnB_ref[...], preferred_element_type=jnp.float32)
        + bnB_ref[...], 0.0)
    r = lax.broadcasted_iota(jnp.int32, (_G, _BLK), 0)
    c = lax.broadcasted_iota(jnp.int32, (_G, _BLK), 1)
    seg = jnp.where(c // S2 == r, 1.0 / S2, 0.0).astype(jnp.float32)
    x1s = jnp.dot(seg, hs, preferred_element_type=jnp.float32)
    x1n = jnp.dot(seg, hn, preferred_element_type=jnp.float32)
    x1_ref[...] = jnp.concatenate([x1s, x1n], axis=1)
    n0_ref[...] = jnp.dot(seg, gh0, preferred_element_type=jnp.float32)


@jax.jit
def _tc_dense1(gh0, gh1sum, wsB, bsB, wnB, bnB):
    grid = (B * S2) // _BLK
    return pl.pallas_call(
        _tc1_body,
        grid=(grid,),
        in_specs=[
            pl.BlockSpec((_BLK, F), lambda i: (i, 0)),
            pl.BlockSpec((_BLK, F), lambda i: (i, 0)),
            pl.BlockSpec((F, H), lambda i: (0, 0)),
            pl.BlockSpec((1, H), lambda i: (0, 0)),
            pl.BlockSpec((F, H), lambda i: (0, 0)),
            pl.BlockSpec((1, H), lambda i: (0, 0)),
        ],
        out_specs=[
            pl.BlockSpec((_G, 2 * H), lambda i: (i, 0)),
            pl.BlockSpec((_G, F), lambda i: (i, 0)),
        ],
        out_shape=[
            jax.ShapeDtypeStruct((B, 2 * H), jnp.float32),
            jax.ShapeDtypeStruct((B, F), jnp.float32),
        ],
    )(gh0, gh1sum, wsB, bsB.reshape(1, H), wnB, bnB.reshape(1, H))


def _tc2_body(g0_ref, n0_ref, x1_ref, wsA_ref, bsA_ref, wnA_ref, bnA_ref,
              w2s_ref, b2s_ref, w2n_ref, b2n_ref, lng_ref, lnb_ref,
              wc_ref, bc_ref, out_ref):
    fs = jnp.maximum(
        jnp.dot(g0_ref[...], wsA_ref[...], preferred_element_type=jnp.float32)
        + bsA_ref[...], 0.0)
    fn = jnp.maximum(
        jnp.dot(n0_ref[...], wnA_ref[...], preferred_element_type=jnp.float32)
        + bnA_ref[...], 0.0)
    fm0 = jnp.concatenate([fs, fn], axis=1)
    p1 = jnp.dot(fm0, w2s_ref[...], preferred_element_type=jnp.float32) \
        + b2s_ref[...]
    p2 = jnp.dot(x1_ref[...], w2n_ref[...], preferred_element_type=jnp.float32) \
        + b2n_ref[...]
    pre = jnp.concatenate([p1, p2], axis=1)
    mu = jnp.mean(pre, axis=-1, keepdims=True)
    d = pre - mu
    var = jnp.mean(d * d, axis=-1, keepdims=True)
    ln = d * lax.rsqrt(var + 1e-12) * lng_ref[...] + lnb_ref[...]
    out_ref[...] = jnp.dot(ln, wc_ref[...],
                           preferred_element_type=jnp.float32) + bc_ref[...]


@jax.jit
def _tc_dense2(g0, n0, x1, wsA, bsA, wnA, bnA, w2s, b2s, w2n, b2n,
               lng, lnb, wc_pad, bc_pad):
    return pl.pallas_call(
        _tc2_body,
        out_shape=jax.ShapeDtypeStruct((B, 128), jnp.float32),
    )(g0, n0, x1, wsA, bsA.reshape(1, H), wnA, bnA.reshape(1, H),
      w2s, b2s.reshape(1, H), w2n, b2n.reshape(1, H),
      lng.reshape(1, 2 * H), lnb.reshape(1, 2 * H), wc_pad,
      bc_pad.reshape(1, 128))


def kernel(x, adj, features, wsA, bsA, wnA, bnA, wsB, bsB, wnB, bnB,
           w2s, b2s, w2n, b2n, lng, lnb, wc, bc):
    x = x.astype(jnp.int32)
    adj = adj.astype(jnp.int32)

    adjp = _tc_build_adjp(adj, jnp.asarray(_SEL))

    a0w = _sc_gather(adjp, x, B, 128, B // _NW)
    hop0f = a0w[:, :S2].reshape(-1)

    a1w = _sc_gather(adjp, hop0f, B * S2, 128, 32)
    hop1f = a1w[:, 32:32 + S1].reshape(-1)

    idx_sf = jnp.concatenate([x, hop0f])
    gf = _sc_gather(features, idx_sf, B + B * S2, F, 32)
    g0 = gf[:B]
    gh0 = gf[B:]

    gh1sum = _sc_gather_sum(features, hop1f, B * S2)

    x1, n0 = _tc_dense1(gh0, gh1sum, wsB, bsB, wnB, bnB)

    wc_pad = jnp.concatenate(
        [wc, jnp.zeros((2 * H, 128 - C), jnp.float32)], axis=1)
    bc_pad = jnp.concatenate([bc, jnp.zeros((128 - C,), jnp.float32)])
    out = _tc_dense2(g0, n0, x1, wsA, bsA, wnA, bnA,
                     w2s, b2s, w2n, b2n, lng, lnb, wc_pad, bc_pad)
    return out[:, :C]


# speedup vs baseline: 2.0108x; 2.0108x over previous
import functools

import numpy as np
import jax
import jax.numpy as jnp
from jax import lax
from jax.experimental import pallas as pl
from jax.experimental.pallas import tpu as pltpu
from jax.experimental.pallas import tpu_sc as plsc

N = 100000
MAXDEG = 32
F = 128
H = 256
S1 = 10
S2 = 25
B = 1024
C = 41

_PERM0 = np.random.RandomState(0).permutation(MAXDEG)
_PERM1 = np.random.RandomState(1).permutation(MAXDEG)

_SEL = np.zeros((MAXDEG, 128), np.float32)
for _i in range(S2):
    _SEL[_PERM0[_i], _i] = 1.0
for _j in range(S1):
    _SEL[_PERM1[_j], 32 + _j] = 1.0

_NC = 2
_NS = 16
_NW = _NC * _NS


def _sc_mesh():
    return plsc.VectorSubcoreMesh(core_axis_name="c", subcore_axis_name="s")


@functools.partial(jax.jit, static_argnums=(2, 3, 4))
def _sc_gather(table, idx, M, D, R):
    m = M // _NW
    nch = m // R
    dtype = table.dtype

    @functools.partial(
        pl.kernel,
        mesh=_sc_mesh(),
        out_type=jax.ShapeDtypeStruct((M, D), dtype),
        scratch_types=[
            pltpu.VMEM((R,), jnp.int32),
            pltpu.VMEM((R,), jnp.int32),
            pltpu.VMEM((R, D), dtype),
            pltpu.VMEM((R, D), dtype),
            pltpu.SemaphoreType.DMA,
            pltpu.SemaphoreType.DMA,
        ],
    )
    def k(table_hbm, idx_hbm, out_hbm, ixa, ixb, bufa, bufb, gsem, ssem):
        wid = lax.axis_index("s") * _NC + lax.axis_index("c")
        base = wid * m

        def pf(c, ix, buf):
            pltpu.sync_copy(idx_hbm.at[pl.ds(base + c * R, R)], ix)
            pltpu.make_async_copy(table_hbm.at[ix], buf, gsem).start()

        def gwait(ix, buf):
            pltpu.make_async_copy(table_hbm.at[ix], buf, gsem).wait()

        def sto(c, buf):
            return pltpu.make_async_copy(
                buf, out_hbm.at[pl.ds(base + c * R, R)], ssem)

        if nch == 1:
            pf(0, ixa, bufa)
            gwait(ixa, bufa)
            pltpu.sync_copy(bufa, out_hbm.at[pl.ds(base, R)])
            return

        nit = nch // 2
        pf(0, ixa, bufa)

        def body(i, carry):
            a = 2 * i
            b = a + 1

            @pl.when(i > 0)
            def _():
                sto(b - 2, bufb).wait()

            pf(b, ixb, bufb)
            gwait(ixa, bufa)
            sto(a, bufa).start()
            gwait(ixb, bufb)
            sto(a, bufa).wait()

            @pl.when(i < nit - 1)
            def _():
                pf(a + 2, ixa, bufa)

            sto(b, bufb).start()
            return carry

        lax.fori_loop(0, nit, body, 0)
        sto(nch - 1, bufb).wait()

    return k(table, idx)


_CT = 4
_RC = _CT * S1


@functools.partial(jax.jit, static_argnums=(2,))
def _sc_gather_sum(table, idx, M2):
    m = M2 // _NW
    nch = m // _CT
    nit = nch // 2

    @functools.partial(
        pl.kernel,
        mesh=_sc_mesh(),
        out_type=jax.ShapeDtypeStruct((M2, F), jnp.float32),
        scratch_types=[
            pltpu.VMEM((_RC,), jnp.int32),
            pltpu.VMEM((_RC,), jnp.int32),
            pltpu.VMEM((_RC, F), jnp.float32),
            pltpu.VMEM((_RC, F), jnp.float32),
            pltpu.VMEM((_CT, F), jnp.float32),
            pltpu.VMEM((_CT, F), jnp.float32),
            pltpu.SemaphoreType.DMA,
            pltpu.SemaphoreType.DMA,
        ],
    )
    def k(table_hbm, idx_hbm, out_hbm, ixa, ixb, bufa, bufb, acca, accb,
          gsem, ssem):
        wid = lax.axis_index("s") * _NC + lax.axis_index("c")
        tbase = wid * m

        def pf(c, ix, buf):
            pltpu.sync_copy(
                idx_hbm.at[pl.ds((tbase + c * _CT) * S1, _RC)], ix)
            pltpu.make_async_copy(table_hbm.at[ix], buf, gsem).start()

        def gwait(ix, buf):
            pltpu.make_async_copy(table_hbm.at[ix], buf, gsem).wait()

        def sto(c, acc):
            return pltpu.make_async_copy(
                acc, out_hbm.at[pl.ds(tbase + c * _CT, _CT)], ssem)

        def reduce(buf, acc):
            for t in range(_CT):
                for v in range(F // 16):
                    sl = pl.ds(v * 16, 16)
                    a = buf[t * S1, sl]
                    for s in range(1, S1):
                        a = a + buf[t * S1 + s, sl]
                    acc[t, sl] = a

        pf(0, ixa, bufa)

        def body(i, carry):
            a = 2 * i
            b = a + 1
            pf(b, ixb, bufb)
            gwait(ixa, bufa)

            @pl.when(i > 0)
            def _():
                sto(a - 2, acca).wait()

            reduce(bufa, acca)
            sto(a, acca).start()

            @pl.when(i < nit - 1)
            def _():
                pf(a + 2, ixa, bufa)

            gwait(ixb, bufb)

            @pl.when(i > 0)
            def _():
                sto(b - 2, accb).wait()

            reduce(bufb, accb)
            sto(b, accb).start()
            return carry

        lax.fori_loop(0, nit, body, 0)
        sto(nch - 2, acca).wait()
        sto(nch - 1, accb).wait()

    return k(table, idx)


_BRB = 2000


def _build_body(adj_ref, sel_ref, out_ref):
    af = adj_ref[...].astype(jnp.float32)
    p = jnp.dot(af, sel_ref[...], preferred_element_type=jnp.float32,
                precision=lax.Precision.HIGHEST)
    out_ref[...] = jnp.round(p).astype(jnp.int32)


@jax.jit
def _tc_build_adjp(adj, sel):
    return pl.pallas_call(
        _build_body,
        grid=(N // _BRB,),
        in_specs=[
            pl.BlockSpec((_BRB, MAXDEG), lambda i: (i, 0)),
            pl.BlockSpec((MAXDEG, 128), lambda i: (0, 0)),
        ],
        out_specs=pl.BlockSpec((_BRB, 128), lambda i: (i, 0)),
        out_shape=jax.ShapeDtypeStruct((N, 128), jnp.int32),
    )(adj, sel)


_BLK = 800
_G = _BLK // S2


def _tc1_body(gh0_ref, gs_ref, wsB_ref, bsB_ref, wnB_ref, bnB_ref,
              x1_ref, n0_ref):
    gh0 = gh0_ref[...]
    gs = gs_ref[...] * (1.0 / S1)
    hs = jnp.maximum(
        jnp.dot(gh0, wsB_ref[...], preferred_element_type=jnp.float32)
        + bsB_ref[...], 0.0)
    hn = jnp.maximum(
        jnp.dot(gs, wnB_ref[...], preferred_element_type=jnp.float32)
        + bnB_ref[...], 0.0)
    r = lax.broadcasted_iota(jnp.int32, (_G, _BLK), 0)
    c = lax.broadcasted_iota(jnp.int32, (_G, _BLK), 1)
    seg = jnp.where(c // S2 == r, 1.0 / S2, 0.0).astype(jnp.float32)
    x1s = jnp.dot(seg, hs, preferred_element_type=jnp.float32)
    x1n = jnp.dot(seg, hn, preferred_element_type=jnp.float32)
    x1_ref[...] = jnp.concatenate([x1s, x1n], axis=1)
    n0_ref[...] = jnp.dot(seg, gh0, preferred_element_type=jnp.float32)


@jax.jit
def _tc_dense1(gh0, gh1sum, wsB, bsB, wnB, bnB):
    grid = (B * S2) // _BLK
    return pl.pallas_call(
        _tc1_body,
        grid=(grid,),
        in_specs=[
            pl.BlockSpec((_BLK, F), lambda i: (i, 0)),
            pl.BlockSpec((_BLK, F), lambda i: (i, 0)),
            pl.BlockSpec((F, H), lambda i: (0, 0)),
            pl.BlockSpec((1, H), lambda i: (0, 0)),
            pl.BlockSpec((F, H), lambda i: (0, 0)),
            pl.BlockSpec((1, H), lambda i: (0, 0)),
        ],
        out_specs=[
            pl.BlockSpec((_G, 2 * H), lambda i: (i, 0)),
            pl.BlockSpec((_G, F), lambda i: (i, 0)),
        ],
        out_shape=[
            jax.ShapeDtypeStruct((B, 2 * H), jnp.float32),
            jax.ShapeDtypeStruct((B, F), jnp.float32),
        ],
    )(gh0, gh1sum, wsB, bsB.reshape(1, H), wnB, bnB.reshape(1, H))


def _tc2_body(g0_ref, n0_ref, x1_ref, wsA_ref, bsA_ref, wnA_ref, bnA_ref,
              w2s_ref, b2s_ref, w2n_ref, b2n_ref, lng_ref, lnb_ref,
              wc_ref, bc_ref, out_ref):
    fs = jnp.maximum(
        jnp.dot(g0_ref[...], wsA_ref[...], preferred_element_type=jnp.float32)
        + bsA_ref[...], 0.0)
    fn = jnp.maximum(
        jnp.dot(n0_ref[...], wnA_ref[...], preferred_element_type=jnp.float32)
        + bnA_ref[...], 0.0)
    fm0 = jnp.concatenate([fs, fn], axis=1)
    p1 = jnp.dot(fm0, w2s_ref[...], preferred_element_type=jnp.float32) \
        + b2s_ref[...]
    p2 = jnp.dot(x1_ref[...], w2n_ref[...], preferred_element_type=jnp.float32) \
        + b2n_ref[...]
    pre = jnp.concatenate([p1, p2], axis=1)
    mu = jnp.mean(pre, axis=-1, keepdims=True)
    d = pre - mu
    var = jnp.mean(d * d, axis=-1, keepdims=True)
    ln = d * lax.rsqrt(var + 1e-12) * lng_ref[...] + lnb_ref[...]
    out_ref[...] = jnp.dot(ln, wc_ref[...],
                           preferred_element_type=jnp.float32) + bc_ref[...]


@jax.jit
def _tc_dense2(g0, n0, x1, wsA, bsA, wnA, bnA, w2s, b2s, w2n, b2n,
               lng, lnb, wc_pad, bc_pad):
    return pl.pallas_call(
        _tc2_body,
        out_shape=jax.ShapeDtypeStruct((B, 128), jnp.float32),
    )(g0, n0, x1, wsA, bsA.reshape(1, H), wnA, bnA.reshape(1, H),
      w2s, b2s.reshape(1, H), w2n, b2n.reshape(1, H),
      lng.reshape(1, 2 * H), lnb.reshape(1, 2 * H), wc_pad,
      bc_pad.reshape(1, 128))


def kernel(x, adj, features, wsA, bsA, wnA, bnA, wsB, bsB, wnB, bnB,
           w2s, b2s, w2n, b2n, lng, lnb, wc, bc):
    x = x.astype(jnp.int32)
    adj = adj.astype(jnp.int32)

    adjp = _tc_build_adjp(adj, jnp.asarray(_SEL))

    a0w = _sc_gather(adjp, x, B, 128, B // _NW)
    hop0f = a0w[:, :S2].reshape(-1)

    a1w = _sc_gather(adjp, hop0f, B * S2, 128, 80)
    hop1f = a1w[:, 32:32 + S1].reshape(-1)

    idx_sf = jnp.concatenate([x, hop0f])
    gf = _sc_gather(features, idx_sf, B + B * S2, F, 32)
    g0 = gf[:B]
    gh0 = gf[B:]

    gh1sum = _sc_gather_sum(features, hop1f, B * S2)

    x1, n0 = _tc_dense1(gh0, gh1sum, wsB, bsB, wnB, bnB)

    wc_pad = jnp.concatenate(
        [wc, jnp.zeros((2 * H, 128 - C), jnp.float32)], axis=1)
    bc_pad = jnp.concatenate([bc, jnp.zeros((128 - C,), jnp.float32)])
    out = _tc_dense2(g0, n0, x1, wsA, bsA, wnA, bnA,
                     w2s, b2s, w2n, b2n, lng, lnb, wc_pad, bc_pad)
    return out[:, :C]


# speedup vs baseline: 2.0767x; 1.0327x over previous
import functools

import numpy as np
import jax
import jax.numpy as jnp
from jax import lax
from jax.experimental import pallas as pl
from jax.experimental.pallas import tpu as pltpu
from jax.experimental.pallas import tpu_sc as plsc

N = 100000
MAXDEG = 32
F = 128
H = 256
S1 = 10
S2 = 25
B = 1024
C = 41

_PERM0 = np.random.RandomState(0).permutation(MAXDEG)
_PERM1 = np.random.RandomState(1).permutation(MAXDEG)

_SEL = np.zeros((MAXDEG, 128), np.float32)
for _i in range(S2):
    _SEL[_PERM0[_i], _i] = 1.0
for _j in range(S1):
    _SEL[_PERM1[_j], 32 + _j] = 1.0
_SELIDX = np.zeros((128,), np.int32)
_SELIDX[:S2] = _PERM0[:S2]
_SELIDX[32:32 + S1] = _PERM1[:S1]

_NC = 2
_NS = 16
_NW = _NC * _NS


def _sc_mesh():
    return plsc.VectorSubcoreMesh(core_axis_name="c", subcore_axis_name="s")


@functools.partial(jax.jit, static_argnums=(2, 3, 4))
def _sc_gather(table, idx, M, D, R):
    m = M // _NW
    nch = m // R
    dtype = table.dtype

    @functools.partial(
        pl.kernel,
        mesh=_sc_mesh(),
        out_type=jax.ShapeDtypeStruct((M, D), dtype),
        scratch_types=[
            pltpu.VMEM((R,), jnp.int32),
            pltpu.VMEM((R,), jnp.int32),
            pltpu.VMEM((R, D), dtype),
            pltpu.VMEM((R, D), dtype),
            pltpu.SemaphoreType.DMA,
            pltpu.SemaphoreType.DMA,
        ],
    )
    def k(table_hbm, idx_hbm, out_hbm, ixa, ixb, bufa, bufb, gsem, ssem):
        wid = lax.axis_index("s") * _NC + lax.axis_index("c")
        base = wid * m

        def pf(c, ix, buf):
            pltpu.sync_copy(idx_hbm.at[pl.ds(base + c * R, R)], ix)
            pltpu.make_async_copy(table_hbm.at[ix], buf, gsem).start()

        def gwait(ix, buf):
            pltpu.make_async_copy(table_hbm.at[ix], buf, gsem).wait()

        def sto(c, buf):
            return pltpu.make_async_copy(
                buf, out_hbm.at[pl.ds(base + c * R, R)], ssem)

        if nch == 1:
            pf(0, ixa, bufa)
            gwait(ixa, bufa)
            pltpu.sync_copy(bufa, out_hbm.at[pl.ds(base, R)])
            return

        nit = nch // 2
        pf(0, ixa, bufa)

        def body(i, carry):
            a = 2 * i
            b = a + 1

            @pl.when(i > 0)
            def _():
                sto(b - 2, bufb).wait()

            pf(b, ixb, bufb)
            gwait(ixa, bufa)
            sto(a, bufa).start()
            gwait(ixb, bufb)
            sto(a, bufa).wait()

            @pl.when(i < nit - 1)
            def _():
                pf(a + 2, ixa, bufa)

            sto(b, bufb).start()
            return carry

        lax.fori_loop(0, nit, body, 0)
        sto(nch - 1, bufb).wait()

    return k(table, idx)


_CT = 4
_RC = _CT * S1


@functools.partial(jax.jit, static_argnums=(2,))
def _sc_gather_sum(table, idx, M2):
    m = M2 // _NW
    nch = m // _CT
    nit = nch // 2

    @functools.partial(
        pl.kernel,
        mesh=_sc_mesh(),
        out_type=jax.ShapeDtypeStruct((M2, F), jnp.float32),
        scratch_types=[
            pltpu.VMEM((_RC,), jnp.int32),
            pltpu.VMEM((_RC,), jnp.int32),
            pltpu.VMEM((_RC, F), jnp.float32),
            pltpu.VMEM((_RC, F), jnp.float32),
            pltpu.VMEM((_CT, F), jnp.float32),
            pltpu.VMEM((_CT, F), jnp.float32),
            pltpu.SemaphoreType.DMA,
            pltpu.SemaphoreType.DMA,
        ],
    )
    def k(table_hbm, idx_hbm, out_hbm, ixa, ixb, bufa, bufb, acca, accb,
          gsem, ssem):
        wid = lax.axis_index("s") * _NC + lax.axis_index("c")
        tbase = wid * m

        def pf(c, ix, buf):
            pltpu.sync_copy(
                idx_hbm.at[pl.ds((tbase + c * _CT) * S1, _RC)], ix)
            pltpu.make_async_copy(table_hbm.at[ix], buf, gsem).start()

        def gwait(ix, buf):
            pltpu.make_async_copy(table_hbm.at[ix], buf, gsem).wait()

        def sto(c, acc):
            return pltpu.make_async_copy(
                acc, out_hbm.at[pl.ds(tbase + c * _CT, _CT)], ssem)

        def reduce(buf, acc):
            for t in range(_CT):
                for v in range(F // 16):
                    sl = pl.ds(v * 16, 16)
                    a = buf[t * S1, sl]
                    for s in range(1, S1):
                        a = a + buf[t * S1 + s, sl]
                    acc[t, sl] = a

        pf(0, ixa, bufa)

        def body(i, carry):
            a = 2 * i
            b = a + 1
            pf(b, ixb, bufb)
            gwait(ixa, bufa)

            @pl.when(i > 0)
            def _():
                sto(a - 2, acca).wait()

            reduce(bufa, acca)
            sto(a, acca).start()

            @pl.when(i < nit - 1)
            def _():
                pf(a + 2, ixa, bufa)

            gwait(ixb, bufb)

            @pl.when(i > 0)
            def _():
                sto(b - 2, accb).wait()

            reduce(bufb, accb)
            sto(b, accb).start()
            return carry

        lax.fori_loop(0, nit, body, 0)
        sto(nch - 2, acca).wait()
        sto(nch - 1, accb).wait()

    return k(table, idx)


_BRB = 2000


def _build_body(adj_ref, sel_ref, out_ref):
    idx = jnp.broadcast_to(sel_ref[...], (_BRB, 128))
    out_ref[...] = jnp.take_along_axis(adj_ref[...], idx, axis=1)


@jax.jit
def _tc_build_adjp(adj, sel):
    return pl.pallas_call(
        _build_body,
        grid=(N // _BRB,),
        in_specs=[
            pl.BlockSpec((_BRB, MAXDEG), lambda i: (i, 0)),
            pl.BlockSpec((1, 128), lambda i: (0, 0)),
        ],
        out_specs=pl.BlockSpec((_BRB, 128), lambda i: (i, 0)),
        out_shape=jax.ShapeDtypeStruct((N, 128), jnp.int32),
    )(adj, sel)


_BLK = 800
_G = _BLK // S2


def _tc1_body(gh0_ref, gs_ref, wsB_ref, bsB_ref, wnB_ref, bnB_ref,
              x1_ref, n0_ref):
    gh0 = gh0_ref[...]
    gs = gs_ref[...] * (1.0 / S1)
    hs = jnp.maximum(
        jnp.dot(gh0, wsB_ref[...], preferred_element_type=jnp.float32)
        + bsB_ref[...], 0.0)
    hn = jnp.maximum(
        jnp.dot(gs, wnB_ref[...], preferred_element_type=jnp.float32)
        + bnB_ref[...], 0.0)
    r = lax.broadcasted_iota(jnp.int32, (_G, _BLK), 0)
    c = lax.broadcasted_iota(jnp.int32, (_G, _BLK), 1)
    seg = jnp.where(c // S2 == r, 1.0 / S2, 0.0).astype(jnp.float32)
    x1s = jnp.dot(seg, hs, preferred_element_type=jnp.float32)
    x1n = jnp.dot(seg, hn, preferred_element_type=jnp.float32)
    x1_ref[...] = jnp.concatenate([x1s, x1n], axis=1)
    n0_ref[...] = jnp.dot(seg, gh0, preferred_element_type=jnp.float32)


@jax.jit
def _tc_dense1(gh0, gh1sum, wsB, bsB, wnB, bnB):
    grid = (B * S2) // _BLK
    return pl.pallas_call(
        _tc1_body,
        grid=(grid,),
        in_specs=[
            pl.BlockSpec((_BLK, F), lambda i: (i, 0)),
            pl.BlockSpec((_BLK, F), lambda i: (i, 0)),
            pl.BlockSpec((F, H), lambda i: (0, 0)),
            pl.BlockSpec((1, H), lambda i: (0, 0)),
            pl.BlockSpec((F, H), lambda i: (0, 0)),
            pl.BlockSpec((1, H), lambda i: (0, 0)),
        ],
        out_specs=[
            pl.BlockSpec((_G, 2 * H), lambda i: (i, 0)),
            pl.BlockSpec((_G, F), lambda i: (i, 0)),
        ],
        out_shape=[
            jax.ShapeDtypeStruct((B, 2 * H), jnp.float32),
            jax.ShapeDtypeStruct((B, F), jnp.float32),
        ],
    )(gh0, gh1sum, wsB, bsB.reshape(1, H), wnB, bnB.reshape(1, H))


def _tc2_body(g0_ref, n0_ref, x1_ref, wsA_ref, bsA_ref, wnA_ref, bnA_ref,
              w2s_ref, b2s_ref, w2n_ref, b2n_ref, lng_ref, lnb_ref,
              wc_ref, bc_ref, out_ref):
    fs = jnp.maximum(
        jnp.dot(g0_ref[...], wsA_ref[...], preferred_element_type=jnp.float32)
        + bsA_ref[...], 0.0)
    fn = jnp.maximum(
        jnp.dot(n0_ref[...], wnA_ref[...], preferred_element_type=jnp.float32)
        + bnA_ref[...], 0.0)
    fm0 = jnp.concatenate([fs, fn], axis=1)
    p1 = jnp.dot(fm0, w2s_ref[...], preferred_element_type=jnp.float32) \
        + b2s_ref[...]
    p2 = jnp.dot(x1_ref[...], w2n_ref[...], preferred_element_type=jnp.float32) \
        + b2n_ref[...]
    pre = jnp.concatenate([p1, p2], axis=1)
    mu = jnp.mean(pre, axis=-1, keepdims=True)
    d = pre - mu
    var = jnp.mean(d * d, axis=-1, keepdims=True)
    ln = d * lax.rsqrt(var + 1e-12) * lng_ref[...] + lnb_ref[...]
    out_ref[...] = jnp.dot(ln, wc_ref[...],
                           preferred_element_type=jnp.float32) + bc_ref[...]


@jax.jit
def _tc_dense2(g0, n0, x1, wsA, bsA, wnA, bnA, w2s, b2s, w2n, b2n,
               lng, lnb, wc_pad, bc_pad):
    return pl.pallas_call(
        _tc2_body,
        out_shape=jax.ShapeDtypeStruct((B, 128), jnp.float32),
    )(g0, n0, x1, wsA, bsA.reshape(1, H), wnA, bnA.reshape(1, H),
      w2s, b2s.reshape(1, H), w2n, b2n.reshape(1, H),
      lng.reshape(1, 2 * H), lnb.reshape(1, 2 * H), wc_pad,
      bc_pad.reshape(1, 128))


def kernel(x, adj, features, wsA, bsA, wnA, bnA, wsB, bsB, wnB, bnB,
           w2s, b2s, w2n, b2n, lng, lnb, wc, bc):
    x = x.astype(jnp.int32)
    adj = adj.astype(jnp.int32)

    adjp = _tc_build_adjp(adj, jnp.asarray(_SELIDX).reshape(1, 128))

    a0w = _sc_gather(adjp, x, B, 128, B // _NW)
    hop0f = a0w[:, :S2].reshape(-1)

    a1w = _sc_gather(adjp, hop0f, B * S2, 128, 80)
    hop1f = a1w[:, 32:32 + S1].reshape(-1)

    idx_sf = jnp.concatenate([x, hop0f])
    gf = _sc_gather(features, idx_sf, B + B * S2, F, 32)
    g0 = gf[:B]
    gh0 = gf[B:]

    gh1sum = _sc_gather_sum(features, hop1f, B * S2)

    x1, n0 = _tc_dense1(gh0, gh1sum, wsB, bsB, wnB, bnB)

    wc_pad = jnp.concatenate(
        [wc, jnp.zeros((2 * H, 128 - C), jnp.float32)], axis=1)
    bc_pad = jnp.concatenate([bc, jnp.zeros((128 - C,), jnp.float32)])
    out = _tc_dense2(g0, n0, x1, wsA, bsA, wnA, bnA,
                     w2s, b2s, w2n, b2n, lng, lnb, wc_pad, bc_pad)
    return out[:, :C]
